# baseline (speedup 1.0000x reference)
import jax
import jax.numpy as jnp
import numpy as np
from jax import lax
from jax.experimental import pallas as pl
from jax.experimental.pallas import tpu as pltpu
from jax.experimental.pallas import tpu_sc as plsc

M = 32
NUM_CODES = 256
EMB_DIM = 16
BATCH = 16384
B_FLAT = BATCH * M
NC, NS = 2, 16
NW = NC * NS
BATCH_PER_W = BATCH // NW
ROWS_PER_W = B_FLAT // NW
G = 128
GPC = 16
CHUNK_ROWS = GPC * G
NCHUNK = ROWS_PER_W // CHUNK_ROWS
L = 16


def _sc_body(codes_hbm, table_hbm, out_hbm,
             codes_v, idx_v, rows_v, shared_tab,
             sem_tab, sem_codes, sem_g, sem_o0, sem_o1):
    sid = lax.axis_index("s")
    wid = sid * NC + lax.axis_index("c")
    base = wid * ROWS_PER_W

    ccopy = pltpu.async_copy(
        codes_hbm.at[pl.ds(wid * BATCH_PER_W, BATCH_PER_W), pl.ds(0, M)],
        codes_v, sem_codes)

    @pl.when(sid == 0)
    def _():
        tcopies = [
            pltpu.make_async_copy(
                table_hbm.at[i],
                shared_tab.at[pl.ds(i * NUM_CODES, NUM_CODES)], sem_tab)
            for i in range(M)
        ]
        for tc in tcopies:
            tc.start()
        for tc in tcopies:
            tc.wait()

    ccopy.wait()
    plsc.subcore_barrier()

    iot = lax.iota(jnp.int32, L)
    lane_lo = lax.bitwise_and(iot, 7)
    lane_hi = lax.shift_right_logical(iot, 3)
    col_ct = [lane_lo + 8 * ct for ct in range(4)]
    off_ct = [lane_lo * NUM_CODES + 2048 * ct for ct in range(4)]
    out_sems = (sem_o0, sem_o1)

    def build_idx(c, slot):
        row_base = lane_hi + 64 * c
        for g in range(GPC):
            rb_l = g // 2
            for q in range(G // L):
                ct = 2 * (g % 2) + q // 4
                row16 = row_base + (8 * rb_l + 2 * (q % 4))
                code16 = plsc.load_gather(codes_v, [row16, col_ct[ct]])
                idx_v[slot, g, pl.ds(q * L, L)] = code16 + off_ct[ct]

    def do_chunk(c, slot):
        @pl.when(c >= 2)
        def _():
            pltpu.make_async_copy(
                rows_v.at[slot], out_hbm.at[pl.ds(base, CHUNK_ROWS)],
                out_sems[slot]).wait()

        copies = [
            pltpu.async_copy(shared_tab.at[idx_v.at[slot, g]],
                             rows_v.at[slot, pl.ds(g * G, G)], sem_g)
            for g in range(GPC)
        ]

        @pl.when(c + 1 < NCHUNK)
        def _():
            build_idx(c + 1, 1 - slot)

        for cp in copies:
            cp.wait()

        pltpu.async_copy(rows_v.at[slot],
                         out_hbm.at[pl.ds(base + c * CHUNK_ROWS, CHUNK_ROWS)],
                         out_sems[slot])

    build_idx(0, 0)

    def pair(i, carry):
        do_chunk(2 * i, 0)
        do_chunk(2 * i + 1, 1)
        return carry

    lax.fori_loop(0, NCHUNK // 2, pair, None)

    for slot in range(2):
        pltpu.make_async_copy(rows_v.at[slot],
                              out_hbm.at[pl.ds(base, CHUNK_ROWS)],
                              out_sems[slot]).wait()


_pq_gather = pl.kernel(
    _sc_body,
    out_type=jax.ShapeDtypeStruct((B_FLAT, EMB_DIM), jnp.float32),
    mesh=plsc.VectorSubcoreMesh(core_axis_name="c", subcore_axis_name="s"),
    compiler_params=pltpu.CompilerParams(use_tc_tiling_on_sc=False,
                                         needs_layout_passes=False),
    scratch_types=[
        pltpu.VMEM((BATCH_PER_W, M), jnp.int32),
        pltpu.VMEM((2, GPC, G), jnp.int32),
        pltpu.VMEM((2, CHUNK_ROWS, EMB_DIM), jnp.float32),
        pltpu.VMEM_SHARED((M * NUM_CODES, EMB_DIM), jnp.float32),
        pltpu.SemaphoreType.DMA,
        pltpu.SemaphoreType.DMA,
        pltpu.SemaphoreType.DMA,
        pltpu.SemaphoreType.DMA,
        pltpu.SemaphoreType.DMA,
    ],
)


def kernel(pq_codes, tables):
    codes_pad = jnp.pad(pq_codes.astype(jnp.int32), ((0, 0), (0, 128 - M)))
    out = _pq_gather(codes_pad, tables)
    d = out.reshape(BATCH // 8, 4, 8, 128)
    return d.transpose(0, 2, 1, 3).reshape(BATCH, M * EMB_DIM)


# speedup vs baseline: 1.0294x; 1.0294x over previous
import jax
import jax.numpy as jnp
import numpy as np
from jax import lax
from jax.experimental import pallas as pl
from jax.experimental.pallas import tpu as pltpu
from jax.experimental.pallas import tpu_sc as plsc

M = 32
NUM_CODES = 256
EMB_DIM = 16
BATCH = 16384
B_FLAT = BATCH * M
NC, NS = 2, 16
NW = NC * NS
BATCH_PER_W = BATCH // NW
ROWS_PER_W = B_FLAT // NW
G = 128
GPC = 16
CHUNK_ROWS = GPC * G
NCHUNK = ROWS_PER_W // CHUNK_ROWS
L = 16


def _sc_body(codes_hbm, table_hbm, out_hbm,
             codes_v, idx_v, rows_v, shared_tab,
             sem_tab, sem_codes, sem_g, sem_o0, sem_o1):
    sid = lax.axis_index("s")
    wid = sid * NC + lax.axis_index("c")
    base = wid * ROWS_PER_W

    ccopy = pltpu.async_copy(
        codes_hbm.at[pl.ds(wid * BATCH_PER_W, BATCH_PER_W), pl.ds(0, M)],
        codes_v, sem_codes)

    @pl.when(sid == 0)
    def _():
        tcopies = [
            pltpu.make_async_copy(
                table_hbm.at[i],
                shared_tab.at[pl.ds(i * NUM_CODES, NUM_CODES)], sem_tab)
            for i in range(M)
        ]
        for tc in tcopies:
            tc.start()
        for tc in tcopies:
            tc.wait()

    ccopy.wait()
    plsc.subcore_barrier()

    iot = lax.iota(jnp.int32, L)
    lane_lo = lax.bitwise_and(iot, 7)
    lane_hi = lax.shift_right_logical(iot, 3)
    col_ct = [lane_lo + 8 * ct for ct in range(4)]
    off_ct = [lane_lo * NUM_CODES + 2048 * ct for ct in range(4)]
    out_sems = (sem_o0, sem_o1)

    def do_chunk(c, slot):
        @pl.when(c >= 2)
        def _():
            pltpu.make_async_copy(
                rows_v.at[slot], out_hbm.at[pl.ds(base, CHUNK_ROWS)],
                out_sems[slot]).wait()

        row_base = lane_hi + 64 * c
        for g in range(GPC):
            rb_l = g // 2
            for q in range(G // L):
                ct = 2 * (g % 2) + q // 4
                row16 = row_base + (8 * rb_l + 2 * (q % 4))
                code16 = plsc.load_gather(codes_v, [row16, col_ct[ct]])
                idx_v[slot, g, pl.ds(q * L, L)] = code16 + off_ct[ct]

        copies = [
            pltpu.async_copy(shared_tab.at[idx_v.at[slot, g]],
                             rows_v.at[slot, pl.ds(g * G, G)], sem_g)
            for g in range(GPC)
        ]
        for cp in copies:
            cp.wait()

        pltpu.async_copy(rows_v.at[slot],
                         out_hbm.at[pl.ds(base + c * CHUNK_ROWS, CHUNK_ROWS)],
                         out_sems[slot])

    def pair(i, carry):
        do_chunk(2 * i, 0)
        do_chunk(2 * i + 1, 1)
        return carry

    lax.fori_loop(0, NCHUNK // 2, pair, None)

    for slot in range(2):
        pltpu.make_async_copy(rows_v.at[slot],
                              out_hbm.at[pl.ds(base, CHUNK_ROWS)],
                              out_sems[slot]).wait()


_pq_gather = pl.kernel(
    _sc_body,
    out_type=jax.ShapeDtypeStruct((B_FLAT, EMB_DIM), jnp.float32),
    mesh=plsc.VectorSubcoreMesh(core_axis_name="c", subcore_axis_name="s"),
    compiler_params=pltpu.CompilerParams(use_tc_tiling_on_sc=False,
                                         needs_layout_passes=False),
    scratch_types=[
        pltpu.VMEM((BATCH_PER_W, M), jnp.int32),
        pltpu.VMEM((2, GPC, G), jnp.int32),
        pltpu.VMEM((2, CHUNK_ROWS, EMB_DIM), jnp.float32),
        pltpu.VMEM_SHARED((M * NUM_CODES, EMB_DIM), jnp.float32),
        pltpu.SemaphoreType.DMA,
        pltpu.SemaphoreType.DMA,
        pltpu.SemaphoreType.DMA,
        pltpu.SemaphoreType.DMA,
        pltpu.SemaphoreType.DMA,
    ],
)


def kernel(pq_codes, tables):
    codes_pad = jnp.pad(pq_codes.astype(jnp.int32), ((0, 0), (0, 128 - M)))
    out = _pq_gather(codes_pad, tables)
    d = out.reshape(BATCH // 8, 4, 8, 128)
    return d.transpose(0, 2, 1, 3).reshape(BATCH, M * EMB_DIM)
